# baseline (speedup 1.0000x reference)
import functools

import jax
import jax.numpy as jnp
from jax import lax
from jax.experimental import pallas as pl
from jax.experimental.pallas import tpu as pltpu
from jax.experimental.pallas import tpu_sc as plsc

_N_BLK = 2048
_SC_CHUNK = 256


def _ret2_kernel(ret_feat_ref, w_ret_t_ref, b_ret_ref, w2_t_ref, b_lin_ref,
                 out_ref):
    rm = jnp.mean(ret_feat_ref[...], axis=1)
    rp = jnp.dot(rm, w_ret_t_ref[...],
                 preferred_element_type=jnp.float32) + b_ret_ref[...]
    out_ref[...] = jnp.dot(rp, w2_t_ref[...],
                           preferred_element_type=jnp.float32) + b_lin_ref[...]


def _matmul_kernel(h_ref, w1_t_ref, out_ref):
    out_ref[...] = jnp.dot(h_ref[...], w1_t_ref[...],
                           preferred_element_type=jnp.float32)


def kernel(h, ret_feat, batch, W_ret, b_ret, W_lin, b_lin):
    n, h_dim = h.shape
    bsz, r, ret_dim = ret_feat.shape
    w1_t = W_lin[:, :h_dim].T
    w2_t = W_lin[:, h_dim:].T

    ret2 = pl.pallas_call(
        _ret2_kernel,
        out_shape=jax.ShapeDtypeStruct((bsz, h_dim), jnp.float32),
    )(ret_feat, W_ret.T, b_ret.reshape(1, h_dim), w2_t,
      b_lin.reshape(1, h_dim))

    nblk = _N_BLK
    hw1 = pl.pallas_call(
        _matmul_kernel,
        grid=(n // nblk,),
        in_specs=[
            pl.BlockSpec((nblk, h_dim), lambda i: (i, 0)),
            pl.BlockSpec((h_dim, h_dim), lambda i: (0, 0)),
        ],
        out_specs=pl.BlockSpec((nblk, h_dim), lambda i: (i, 0)),
        out_shape=jax.ShapeDtypeStruct((n, h_dim), jnp.float32),
    )(h, w1_t)

    info = plsc.get_sparse_core_info()
    nw = info.num_cores * info.num_subcores
    chunk = _SC_CHUNK
    n_chunks = n // (nw * chunk)
    batch3 = batch.reshape(nw, n_chunks, chunk)
    mesh = plsc.VectorSubcoreMesh(core_axis_name="c", subcore_axis_name="s")

    @functools.partial(
        pl.kernel, mesh=mesh,
        out_type=jax.ShapeDtypeStruct((n, h_dim), jnp.float32),
        scratch_types=[
            pltpu.VMEM((chunk,), jnp.int32),
            pltpu.VMEM((chunk, h_dim), jnp.float32),
            pltpu.SemaphoreType.DMA,
        ],
    )
    def _sc_gather_add(hw1_hbm, batch_hbm, ret2_hbm, out_hbm, idx_v, acc_v,
                       sem):
        wid = lax.axis_index("s") * info.num_cores + lax.axis_index("c")
        for c in range(n_chunks):
            base = wid * n_chunks * chunk + c * chunk
            pltpu.sync_copy(batch_hbm.at[wid, c], idx_v)
            pltpu.sync_copy(hw1_hbm.at[pl.ds(base, chunk)], acc_v)
            pltpu.async_copy(ret2_hbm.at[idx_v], acc_v, sem, add=True).wait()
            pltpu.sync_copy(acc_v, out_hbm.at[pl.ds(base, chunk)])

    return _sc_gather_add(hw1, batch3, ret2)


# speedup vs baseline: 7.5885x; 7.5885x over previous
import functools

import jax
import jax.numpy as jnp
from jax import lax
from jax.experimental import pallas as pl

_N_BLK = 2048


def _fused_kernel(batch_ref, h_ref, w1_t_ref, ret_feat_ref, w_ret_t_ref,
                  b_ret_ref, w2_t_ref, b_lin_ref, out_ref, *, nb, b):
    rm = jnp.mean(ret_feat_ref[...], axis=1)
    rp = jnp.dot(rm, w_ret_t_ref[...],
                 preferred_element_type=jnp.float32) + b_ret_ref[...]
    ret2 = jnp.dot(rp, w2_t_ref[...],
                   preferred_element_type=jnp.float32) + b_lin_ref[...]
    idx = batch_ref[0, :]
    oh = (idx[:, None] == lax.broadcasted_iota(jnp.int32, (nb, b), 1)
          ).astype(jnp.float32)
    out_ref[...] = (
        jnp.dot(h_ref[...], w1_t_ref[...],
                preferred_element_type=jnp.float32)
        + jnp.dot(oh, ret2, preferred_element_type=jnp.float32))


def kernel(h, ret_feat, batch, W_ret, b_ret, W_lin, b_lin):
    n, h_dim = h.shape
    bsz, r, ret_dim = ret_feat.shape
    w1_t = W_lin[:, :h_dim].T
    w2_t = W_lin[:, h_dim:].T

    nblk = _N_BLK
    grid = n // nblk
    batch3 = batch.reshape(grid, 1, nblk)
    zero = lambda i: (0, 0)
    out = pl.pallas_call(
        functools.partial(_fused_kernel, nb=nblk, b=bsz),
        grid=(grid,),
        in_specs=[
            pl.BlockSpec((None, 1, nblk), lambda i: (i, 0, 0)),
            pl.BlockSpec((nblk, h_dim), lambda i: (i, 0)),
            pl.BlockSpec((h_dim, h_dim), zero),
            pl.BlockSpec((bsz, r, ret_dim), lambda i: (0, 0, 0)),
            pl.BlockSpec((h_dim, h_dim), zero),
            pl.BlockSpec((1, h_dim), zero),
            pl.BlockSpec((h_dim, h_dim), zero),
            pl.BlockSpec((1, h_dim), zero),
        ],
        out_specs=pl.BlockSpec((nblk, h_dim), lambda i: (i, 0)),
        out_shape=jax.ShapeDtypeStruct((n, h_dim), jnp.float32),
    )(batch3, h, w1_t, ret_feat, W_ret.T, b_ret.reshape(1, h_dim), w2_t,
      b_lin.reshape(1, h_dim))
    return out


# speedup vs baseline: 9.6351x; 1.2697x over previous
import functools

import jax
import jax.numpy as jnp
from jax import lax
from jax.experimental import pallas as pl

_N_BLK = 4096


def _fused_kernel(batch_ref, h_ref, w1_t_ref, ret_feat_ref, w_ret_t_ref,
                  b_ret_ref, w2_t_ref, b_lin_ref, out_ref, *, nb, b):
    rm = jnp.mean(ret_feat_ref[...], axis=1)
    rp = jnp.dot(rm, w_ret_t_ref[...],
                 preferred_element_type=jnp.float32) + b_ret_ref[...]
    ret2 = jnp.dot(rp, w2_t_ref[...],
                   preferred_element_type=jnp.float32) + b_lin_ref[...]
    idx = batch_ref[0, :]
    oh = (idx[:, None] == lax.broadcasted_iota(jnp.int32, (nb, b), 1)
          ).astype(jnp.float32)
    out_ref[...] = (
        jnp.dot(h_ref[...], w1_t_ref[...],
                preferred_element_type=jnp.float32)
        + jnp.dot(oh, ret2, preferred_element_type=jnp.float32))


def kernel(h, ret_feat, batch, W_ret, b_ret, W_lin, b_lin):
    n, h_dim = h.shape
    bsz, r, ret_dim = ret_feat.shape
    w1_t = W_lin[:, :h_dim].T
    w2_t = W_lin[:, h_dim:].T

    nblk = _N_BLK
    grid = n // nblk
    batch3 = batch.reshape(grid, 1, nblk)
    zero = lambda i: (0, 0)
    out = pl.pallas_call(
        functools.partial(_fused_kernel, nb=nblk, b=bsz),
        grid=(grid,),
        in_specs=[
            pl.BlockSpec((None, 1, nblk), lambda i: (i, 0, 0)),
            pl.BlockSpec((nblk, h_dim), lambda i: (i, 0)),
            pl.BlockSpec((h_dim, h_dim), zero),
            pl.BlockSpec((bsz, r, ret_dim), lambda i: (0, 0, 0)),
            pl.BlockSpec((h_dim, h_dim), zero),
            pl.BlockSpec((1, h_dim), zero),
            pl.BlockSpec((h_dim, h_dim), zero),
            pl.BlockSpec((1, h_dim), zero),
        ],
        out_specs=pl.BlockSpec((nblk, h_dim), lambda i: (i, 0)),
        out_shape=jax.ShapeDtypeStruct((n, h_dim), jnp.float32),
    )(batch3, h, w1_t, ret_feat, W_ret.T, b_ret.reshape(1, h_dim), w2_t,
      b_lin.reshape(1, h_dim))
    return out


# speedup vs baseline: 10.7968x; 1.1206x over previous
import functools

import jax
import jax.numpy as jnp
from jax import lax
from jax.experimental import pallas as pl

_N_BLK = 8192


def _fused_kernel(batch_ref, h_ref, w1_t_ref, ret_feat_ref, w_ret_t_ref,
                  b_ret_ref, w2_t_ref, b_lin_ref, out_ref, *, nb, b):
    rm = jnp.mean(ret_feat_ref[...], axis=1)
    rp = jnp.dot(rm, w_ret_t_ref[...],
                 preferred_element_type=jnp.float32) + b_ret_ref[...]
    ret2 = jnp.dot(rp, w2_t_ref[...],
                   preferred_element_type=jnp.float32) + b_lin_ref[...]
    idx = batch_ref[0, :]
    oh = (idx[:, None] == lax.broadcasted_iota(jnp.int32, (nb, b), 1)
          ).astype(jnp.float32)
    out_ref[...] = (
        jnp.dot(h_ref[...], w1_t_ref[...],
                preferred_element_type=jnp.float32)
        + jnp.dot(oh, ret2, preferred_element_type=jnp.float32))


def kernel(h, ret_feat, batch, W_ret, b_ret, W_lin, b_lin):
    n, h_dim = h.shape
    bsz, r, ret_dim = ret_feat.shape
    w1_t = W_lin[:, :h_dim].T
    w2_t = W_lin[:, h_dim:].T

    nblk = _N_BLK
    grid = n // nblk
    batch3 = batch.reshape(grid, 1, nblk)
    zero = lambda i: (0, 0)
    out = pl.pallas_call(
        functools.partial(_fused_kernel, nb=nblk, b=bsz),
        grid=(grid,),
        in_specs=[
            pl.BlockSpec((None, 1, nblk), lambda i: (i, 0, 0)),
            pl.BlockSpec((nblk, h_dim), lambda i: (i, 0)),
            pl.BlockSpec((h_dim, h_dim), zero),
            pl.BlockSpec((bsz, r, ret_dim), lambda i: (0, 0, 0)),
            pl.BlockSpec((h_dim, h_dim), zero),
            pl.BlockSpec((1, h_dim), zero),
            pl.BlockSpec((h_dim, h_dim), zero),
            pl.BlockSpec((1, h_dim), zero),
        ],
        out_specs=pl.BlockSpec((nblk, h_dim), lambda i: (i, 0)),
        out_shape=jax.ShapeDtypeStruct((n, h_dim), jnp.float32),
    )(batch3, h, w1_t, ret_feat, W_ret.T, b_ret.reshape(1, h_dim), w2_t,
      b_lin.reshape(1, h_dim))
    return out


# speedup vs baseline: 11.7119x; 1.0848x over previous
import functools

import jax
import jax.numpy as jnp
from jax import lax
from jax.experimental import pallas as pl

_N_BLK = 16384


def _fused_kernel(batch_ref, h_ref, w1_t_ref, ret_feat_ref, w_ret_t_ref,
                  b_ret_ref, w2_t_ref, b_lin_ref, out_ref, *, nb, b):
    rm = jnp.mean(ret_feat_ref[...], axis=1)
    rp = jnp.dot(rm, w_ret_t_ref[...],
                 preferred_element_type=jnp.float32) + b_ret_ref[...]
    ret2 = jnp.dot(rp, w2_t_ref[...],
                   preferred_element_type=jnp.float32) + b_lin_ref[...]
    idx = batch_ref[0, :]
    oh = (idx[:, None] == lax.broadcasted_iota(jnp.int32, (nb, b), 1)
          ).astype(jnp.float32)
    out_ref[...] = (
        jnp.dot(h_ref[...], w1_t_ref[...],
                preferred_element_type=jnp.float32)
        + jnp.dot(oh, ret2, preferred_element_type=jnp.float32))


def kernel(h, ret_feat, batch, W_ret, b_ret, W_lin, b_lin):
    n, h_dim = h.shape
    bsz, r, ret_dim = ret_feat.shape
    w1_t = W_lin[:, :h_dim].T
    w2_t = W_lin[:, h_dim:].T

    nblk = _N_BLK
    grid = n // nblk
    batch3 = batch.reshape(grid, 1, nblk)
    zero = lambda i: (0, 0)
    out = pl.pallas_call(
        functools.partial(_fused_kernel, nb=nblk, b=bsz),
        grid=(grid,),
        in_specs=[
            pl.BlockSpec((None, 1, nblk), lambda i: (i, 0, 0)),
            pl.BlockSpec((nblk, h_dim), lambda i: (i, 0)),
            pl.BlockSpec((h_dim, h_dim), zero),
            pl.BlockSpec((bsz, r, ret_dim), lambda i: (0, 0, 0)),
            pl.BlockSpec((h_dim, h_dim), zero),
            pl.BlockSpec((1, h_dim), zero),
            pl.BlockSpec((h_dim, h_dim), zero),
            pl.BlockSpec((1, h_dim), zero),
        ],
        out_specs=pl.BlockSpec((nblk, h_dim), lambda i: (i, 0)),
        out_shape=jax.ShapeDtypeStruct((n, h_dim), jnp.float32),
    )(batch3, h, w1_t, ret_feat, W_ret.T, b_ret.reshape(1, h_dim), w2_t,
      b_lin.reshape(1, h_dim))
    return out
